# baseline (speedup 1.0000x reference)
import functools

import jax
import jax.numpy as jnp
from jax import lax
from jax.experimental import pallas as pl
from jax.experimental.pallas import tpu as pltpu
from jax.experimental.pallas import tpu_sc as plsc

_L = 512
_R = 1024
_V = 1024
_NK = 3125
_SUB = 256
_NSUB = _R // _SUB
_NW = 32
_SPW = _L // _NW


def _body(ep_hbm, epsub_hbm, rm4_hbm, res_counts_hbm, emb_hbm, sw_hbm, out_hbm,
          ep_v, idx_v, cnt_v, rows_v, emb_v, sw_v, out_v, sem, sem2):
    cid = lax.axis_index("c")
    sid = lax.axis_index("s")
    wid = sid * 2 + cid
    base = wid * _SPW

    ep_cp = pltpu.async_copy(ep_hbm.at[pl.ds(base, _SPW)], ep_v, sem)
    idx_cp = pltpu.async_copy(epsub_hbm.at[wid], idx_v, sem2)
    emb_cp = pltpu.async_copy(emb_hbm, emb_v, sem2)
    sw_cp = pltpu.async_copy(sw_hbm.at[pl.ds(base, _SPW)], sw_v, sem2)
    ep_cp.wait()
    cnt_cp = pltpu.async_copy(res_counts_hbm.at[ep_v], cnt_v, sem)
    idx_cp.wait()
    rows_cp = pltpu.async_copy(
        rm4_hbm.at[idx_v.at[0]], rows_v.at[pl.ds(0, _SPW)], sem)
    cnt_cp.wait()
    rows_cp.wait()

    lanes = lax.iota(jnp.int32, 16)
    cnt = cnt_v[...]
    max_cnt = jnp.max(cnt)

    @pl.when(max_cnt > _SUB)
    def _fetch_tail():
        cps = [
            pltpu.async_copy(
                rm4_hbm.at[idx_v.at[k]], rows_v.at[pl.ds(k * _SPW, _SPW)], sem)
            for k in range(1, _NSUB)
        ]
        for cp in cps:
            cp.wait()

    avg_v = jnp.zeros((16,), jnp.float32)
    for si in range(_SPW):
        cnt_s = cnt[si]
        nch = jnp.minimum((cnt_s + 15) >> 4, _SUB // 16)

        def chunk(j, acc, si=si, cnt_s=cnt_s):
            idx = rows_v[si, pl.ds(j * 16, 16)]
            vals = plsc.load_gather(emb_v, [idx])
            m = (j * 16 + lanes) < cnt_s
            return acc + jnp.where(m, vals, jnp.float32(0.0))

        acc = lax.fori_loop(0, nch, chunk, jnp.zeros((16,), jnp.float32))
        avg_v = jnp.where(lanes == si, jnp.sum(acc), avg_v)

    @pl.when(max_cnt > _SUB)
    def _wide():
        out_v[...] = jnp.zeros((16,), jnp.float32)
        for si in range(_SPW):
            cnt_s = cnt[si]

            @pl.when(cnt_s > _SUB)
            def _site_wide(si=si):
                for k in range(1, _NSUB):
                    def chunk(j, acc, k=k, si=si):
                        idx = rows_v[k * _SPW + si, pl.ds(j * 16, 16)]
                        return acc + plsc.load_gather(emb_v, [idx])

                    acc = lax.fori_loop(
                        0, _SUB // 16, chunk, jnp.zeros((16,), jnp.float32))
                    out_v[...] = out_v[...] + jnp.where(
                        lanes == si, jnp.sum(acc), jnp.float32(0.0))

    @pl.when(max_cnt <= _SUB)
    def _no_wide():
        out_v[...] = jnp.zeros((16,), jnp.float32)

    total = avg_v + out_v[...]
    avg = total / cnt.astype(jnp.float32)
    out_v[...] = jnp.exp(avg + sw_v[...])
    pltpu.sync_copy(out_v, out_hbm.at[pl.ds(base, _SPW)])


@jax.jit
def _run(encoded_parent, epsub, rm4, res_counts, emb, sw):
    mesh = plsc.VectorSubcoreMesh(core_axis_name="c", subcore_axis_name="s")
    f = functools.partial(
        pl.kernel,
        out_type=jax.ShapeDtypeStruct((_L,), jnp.float32),
        mesh=mesh,
        compiler_params=pltpu.CompilerParams(needs_layout_passes=False),
        scratch_types=[
            pltpu.VMEM((_SPW,), jnp.int32),
            pltpu.VMEM((_NSUB, _SPW), jnp.int32),
            pltpu.VMEM((_SPW,), jnp.int32),
            pltpu.VMEM((_NSUB * _SPW, _SUB), jnp.int32),
            pltpu.VMEM((_V,), jnp.float32),
            pltpu.VMEM((_SPW,), jnp.float32),
            pltpu.VMEM((_SPW,), jnp.float32),
            pltpu.SemaphoreType.DMA,
            pltpu.SemaphoreType.DMA,
        ],
    )(_body)
    return f(encoded_parent, epsub, rm4, res_counts, emb, sw)


def kernel(encoded_parent, kmer_emb, site_w, res_map, res_counts):
    emb = kmer_emb.reshape(-1)
    sw = site_w.reshape(-1)
    rm4 = res_map.reshape(_NK * _NSUB, _SUB)
    epsub = (encoded_parent.reshape(_NW, 1, _SPW) * _NSUB
             + jnp.arange(_NSUB, dtype=jnp.int32).reshape(1, _NSUB, 1))
    return _run(encoded_parent, epsub, rm4, res_counts, emb, sw)


# speedup vs baseline: 1.5430x; 1.5430x over previous
import functools

import jax
import jax.numpy as jnp
from jax import lax
from jax.experimental import pallas as pl
from jax.experimental.pallas import tpu as pltpu
from jax.experimental.pallas import tpu_sc as plsc

_L = 512
_R = 1024
_V = 1024
_NK = 3125
_NW = 32
_SPW = _L // _NW


def _body(ep_hbm, res_map_hbm, res_counts_hbm, emb_hbm, sw_hbm, out_hbm,
          ep_v, cnt_v, rows_v, emb_v, sw_v, out_v, sem, sem2):
    cid = lax.axis_index("c")
    sid = lax.axis_index("s")
    wid = sid * 2 + cid
    base = wid * _SPW

    ep_cp = pltpu.async_copy(ep_hbm.at[pl.ds(base, _SPW)], ep_v, sem)
    emb_cp = pltpu.async_copy(emb_hbm, emb_v, sem2)
    sw_cp = pltpu.async_copy(sw_hbm.at[pl.ds(base, _SPW)], sw_v, sem2)
    ep_cp.wait()
    cnt_cp = pltpu.async_copy(res_counts_hbm.at[ep_v], cnt_v, sem)
    rows_cp = pltpu.async_copy(res_map_hbm.at[ep_v], rows_v, sem)
    cnt_cp.wait()
    rows_cp.wait()

    lanes = lax.iota(jnp.int32, 16)
    cnt = cnt_v[...]
    max_cnt = jnp.max(cnt)

    sums = jnp.zeros((16,), jnp.float32)
    for j in range(16):
        row_vals = plsc.load_gather(rows_v, [lanes, jnp.full((16,), j, jnp.int32)])
        vals = plsc.load_gather(emb_v, [row_vals])
        sums = sums + jnp.where(j < cnt, vals, jnp.float32(0.0))

    out_v[...] = jnp.zeros((16,), jnp.float32)

    @pl.when(max_cnt > 16)
    def _tail():
        for si in range(_SPW):
            cnt_s = cnt[si]

            @pl.when(cnt_s > 16)
            def _site(si=si, cnt_s=cnt_s):
                nch = (cnt_s + 15) >> 4

                def chunk(j, acc, si=si, cnt_s=cnt_s):
                    idx = rows_v[si, pl.ds(j * 16, 16)]
                    vals = plsc.load_gather(emb_v, [idx])
                    m = (j * 16 + lanes) < cnt_s
                    return acc + jnp.where(m, vals, jnp.float32(0.0))

                acc = lax.fori_loop(1, nch, chunk, jnp.zeros((16,), jnp.float32))
                out_v[...] = out_v[...] + jnp.where(
                    lanes == si, jnp.sum(acc), jnp.float32(0.0))

    total = sums + out_v[...]
    avg_v = total / cnt.astype(jnp.float32)
    out_v[...] = jnp.exp(avg_v + sw_v[...])
    pltpu.sync_copy(out_v, out_hbm.at[pl.ds(base, _SPW)])


@jax.jit
def _run(encoded_parent, res_map, res_counts, emb, sw):
    mesh = plsc.VectorSubcoreMesh(core_axis_name="c", subcore_axis_name="s")
    f = functools.partial(
        pl.kernel,
        out_type=jax.ShapeDtypeStruct((_L,), jnp.float32),
        mesh=mesh,
        compiler_params=pltpu.CompilerParams(needs_layout_passes=False),
        scratch_types=[
            pltpu.VMEM((_SPW,), jnp.int32),
            pltpu.VMEM((_SPW,), jnp.int32),
            pltpu.VMEM((_SPW, _R), jnp.int32),
            pltpu.VMEM((_V,), jnp.float32),
            pltpu.VMEM((_SPW,), jnp.float32),
            pltpu.VMEM((_SPW,), jnp.float32),
            pltpu.SemaphoreType.DMA,
            pltpu.SemaphoreType.DMA,
        ],
    )(_body)
    return f(encoded_parent, res_map, res_counts, emb, sw)


def kernel(encoded_parent, kmer_emb, site_w, res_map, res_counts):
    emb = kmer_emb.reshape(-1)
    sw = site_w.reshape(-1)
    return _run(encoded_parent, res_map, res_counts, emb, sw)


# speedup vs baseline: 1.5637x; 1.0134x over previous
import functools

import jax
import jax.numpy as jnp
from jax import lax
from jax.experimental import pallas as pl
from jax.experimental.pallas import tpu as pltpu
from jax.experimental.pallas import tpu_sc as plsc

_L = 512
_R = 1024
_V = 1024
_NK = 3125
_NW = 32
_SPW = _L // _NW


def _body(ep_hbm, res_map_hbm, res_counts_hbm, emb_hbm, sw_hbm, out_hbm,
          ep_v, epa_v, epb_v, cnt_v, rowsa_v, rowsb_v, emb_v, sw_v, out_v,
          sem_ep, sem_epa, sem_epb, sem_cnt, sem_ra, sem_rb, sem_io):
    cid = lax.axis_index("c")
    sid = lax.axis_index("s")
    wid = sid * 2 + cid
    base = wid * _SPW

    ep_cp = pltpu.async_copy(ep_hbm.at[pl.ds(base, _SPW)], ep_v, sem_ep)
    epa_cp = pltpu.async_copy(ep_hbm.at[pl.ds(base, 8)], epa_v, sem_epa)
    epb_cp = pltpu.async_copy(ep_hbm.at[pl.ds(base + 8, 8)], epb_v, sem_epb)
    emb_cp = pltpu.async_copy(emb_hbm, emb_v, sem_io)
    sw_cp = pltpu.async_copy(sw_hbm.at[pl.ds(base, _SPW)], sw_v, sem_io)
    epa_cp.wait()
    rows_a = pltpu.async_copy(res_map_hbm.at[epa_v], rowsa_v, sem_ra)
    epb_cp.wait()
    rows_b = pltpu.async_copy(res_map_hbm.at[epb_v], rowsb_v, sem_rb)
    ep_cp.wait()
    cnt_cp = pltpu.async_copy(res_counts_hbm.at[ep_v], cnt_v, sem_cnt)
    emb_cp.wait()
    sw_cp.wait()
    cnt_cp.wait()
    rows_a.wait()

    lanes = lax.iota(jnp.int32, 16)
    cnt = cnt_v[...]
    max_cnt = jnp.max(cnt)

    def _row(si):
        return rowsa_v if si < 8 else rowsb_v

    sums = jnp.zeros((16,), jnp.float32)
    for si in range(8):
        vals = plsc.load_gather(emb_v, [rowsa_v[si, pl.ds(0, 16)]])
        part = jnp.where(lanes < cnt[si], vals, jnp.float32(0.0))
        sums = jnp.where(lanes == si, jnp.sum(part), sums)
    rows_b.wait()
    for si in range(8, _SPW):
        vals = plsc.load_gather(emb_v, [rowsb_v[si - 8, pl.ds(0, 16)]])
        part = jnp.where(lanes < cnt[si], vals, jnp.float32(0.0))
        sums = jnp.where(lanes == si, jnp.sum(part), sums)

    out_v[...] = jnp.zeros((16,), jnp.float32)

    @pl.when(max_cnt > 16)
    def _tail():
        for si in range(_SPW):
            cnt_s = cnt[si]

            @pl.when(cnt_s > 16)
            def _site(si=si, cnt_s=cnt_s):
                nch = (cnt_s + 15) >> 4

                def chunk(j, acc, si=si, cnt_s=cnt_s):
                    idx = _row(si)[si % 8, pl.ds(j * 16, 16)]
                    vals = plsc.load_gather(emb_v, [idx])
                    m = (j * 16 + lanes) < cnt_s
                    return acc + jnp.where(m, vals, jnp.float32(0.0))

                acc = lax.fori_loop(1, nch, chunk, jnp.zeros((16,), jnp.float32))
                out_v[...] = out_v[...] + jnp.where(
                    lanes == si, jnp.sum(acc), jnp.float32(0.0))

    total = sums + out_v[...]
    avg_v = total / cnt.astype(jnp.float32)
    out_v[...] = jnp.exp(avg_v + sw_v[...])
    pltpu.sync_copy(out_v, out_hbm.at[pl.ds(base, _SPW)])


@jax.jit
def _run(encoded_parent, res_map, res_counts, emb, sw):
    mesh = plsc.VectorSubcoreMesh(core_axis_name="c", subcore_axis_name="s")
    f = functools.partial(
        pl.kernel,
        out_type=jax.ShapeDtypeStruct((_L,), jnp.float32),
        mesh=mesh,
        compiler_params=pltpu.CompilerParams(needs_layout_passes=False),
        scratch_types=[
            pltpu.VMEM((_SPW,), jnp.int32),
            pltpu.VMEM((8,), jnp.int32),
            pltpu.VMEM((8,), jnp.int32),
            pltpu.VMEM((_SPW,), jnp.int32),
            pltpu.VMEM((8, _R), jnp.int32),
            pltpu.VMEM((8, _R), jnp.int32),
            pltpu.VMEM((_V,), jnp.float32),
            pltpu.VMEM((_SPW,), jnp.float32),
            pltpu.VMEM((_SPW,), jnp.float32),
            pltpu.SemaphoreType.DMA,
            pltpu.SemaphoreType.DMA,
            pltpu.SemaphoreType.DMA,
            pltpu.SemaphoreType.DMA,
            pltpu.SemaphoreType.DMA,
            pltpu.SemaphoreType.DMA,
            pltpu.SemaphoreType.DMA,
        ],
    )(_body)
    return f(encoded_parent, res_map, res_counts, emb, sw)


def kernel(encoded_parent, kmer_emb, site_w, res_map, res_counts):
    emb = kmer_emb.reshape(-1)
    sw = site_w.reshape(-1)
    return _run(encoded_parent, res_map, res_counts, emb, sw)


# speedup vs baseline: 1.6753x; 1.0714x over previous
import functools

import jax
import jax.numpy as jnp
from jax import lax
from jax.experimental import pallas as pl
from jax.experimental.pallas import tpu as pltpu
from jax.experimental.pallas import tpu_sc as plsc

_L = 512
_R = 1024
_V = 1024
_NK = 3125
_NW = 32
_SPW = _L // _NW


def _body(ep_hbm, res_map_hbm, res_counts_hbm, emb_hbm, sw_hbm, out_hbm,
          ep_v, cnt_v, rows_v, emb_v, sw_v, out_v,
          sem_ep, sem_cnt, sem_rows, sem_io):
    cid = lax.axis_index("c")
    sid = lax.axis_index("s")
    wid = sid * 2 + cid
    base = wid * _SPW

    ep_cp = pltpu.async_copy(ep_hbm.at[pl.ds(base, _SPW)], ep_v, sem_ep)
    emb_cp = pltpu.async_copy(emb_hbm, emb_v, sem_io)
    sw_cp = pltpu.async_copy(sw_hbm.at[pl.ds(base, _SPW)], sw_v, sem_io)
    ep_cp.wait()
    cnt_cp = pltpu.async_copy(res_counts_hbm.at[ep_v], cnt_v, sem_cnt)
    rows_cp = pltpu.async_copy(res_map_hbm.at[ep_v], rows_v, sem_rows)
    emb_cp.wait()
    sw_cp.wait()
    cnt_cp.wait()
    rows_cp.wait()

    lanes = lax.iota(jnp.int32, 16)
    cnt = cnt_v[...]

    def site(si, sums):
        cnt_b = plsc.load_gather(cnt_v, [jnp.full((16,), si, jnp.int32)])
        cnt_s = jnp.max(cnt_b)
        nch = (cnt_s + 15) >> 4

        def chunk(j, acc):
            idx = rows_v[si, pl.ds(j * 16, 16)]
            vals = plsc.load_gather(emb_v, [idx])
            m = (j * 16 + lanes) < cnt_s
            return acc + jnp.where(m, vals, jnp.float32(0.0))

        acc = lax.fori_loop(0, nch, chunk, jnp.zeros((16,), jnp.float32))
        return jnp.where(lanes == si, jnp.sum(acc), sums)

    sums = lax.fori_loop(0, _SPW, site, jnp.zeros((16,), jnp.float32))

    avg_v = sums / cnt.astype(jnp.float32)
    out_v[...] = jnp.exp(avg_v + sw_v[...])
    pltpu.sync_copy(out_v, out_hbm.at[pl.ds(base, _SPW)])


@jax.jit
def _run(encoded_parent, res_map, res_counts, emb, sw):
    mesh = plsc.VectorSubcoreMesh(core_axis_name="c", subcore_axis_name="s")
    f = functools.partial(
        pl.kernel,
        out_type=jax.ShapeDtypeStruct((_L,), jnp.float32),
        mesh=mesh,
        compiler_params=pltpu.CompilerParams(needs_layout_passes=False),
        scratch_types=[
            pltpu.VMEM((_SPW,), jnp.int32),
            pltpu.VMEM((_SPW,), jnp.int32),
            pltpu.VMEM((_SPW, _R), jnp.int32),
            pltpu.VMEM((_V,), jnp.float32),
            pltpu.VMEM((_SPW,), jnp.float32),
            pltpu.VMEM((_SPW,), jnp.float32),
            pltpu.SemaphoreType.DMA,
            pltpu.SemaphoreType.DMA,
            pltpu.SemaphoreType.DMA,
            pltpu.SemaphoreType.DMA,
        ],
    )(_body)
    return f(encoded_parent, res_map, res_counts, emb, sw)


def kernel(encoded_parent, kmer_emb, site_w, res_map, res_counts):
    emb = kmer_emb.reshape(-1)
    sw = site_w.reshape(-1)
    return _run(encoded_parent, res_map, res_counts, emb, sw)
